# baseline (speedup 1.0000x reference)
import functools

import jax
import jax.numpy as jnp
from jax.experimental import pallas as pl
from jax.experimental.pallas import tpu as pltpu

_GUMBEL_SEED = 1234
_INT_MAX = 2**31 - 1


def _gumbel_argmax_onehot(x_ref, w_ref, b_ref, g_ref, out_ref,
                          rmax_ref, ridx_ref, *, tn, vocab):
    p = pl.program_id(0)
    j = pl.program_id(1)

    @pl.when(p == 0)
    def _scan():
        vals = jnp.dot(x_ref[...], w_ref[...],
                       preferred_element_type=jnp.float32)
        vals = vals + b_ref[...] + g_ref[...]
        col = jax.lax.broadcasted_iota(jnp.int32, vals.shape, 1) + j * tn
        vals = jnp.where(col < vocab, vals, -jnp.inf)
        local_max = jnp.max(vals, axis=1, keepdims=True)
        cand = jnp.where(vals == local_max, col, _INT_MAX)
        local_arg = jnp.min(cand, axis=1, keepdims=True)

        @pl.when(j == 0)
        def _init():
            rmax_ref[...] = local_max
            ridx_ref[...] = local_arg

        @pl.when(j > 0)
        def _update():
            better = local_max > rmax_ref[...]
            rmax_ref[...] = jnp.where(better, local_max, rmax_ref[...])
            ridx_ref[...] = jnp.where(better, local_arg, ridx_ref[...])

    @pl.when(p == 1)
    def _write():
        m = out_ref.shape[0]
        col = jax.lax.broadcasted_iota(jnp.int32, (m, tn), 1) + j * tn
        out_ref[...] = (col == ridx_ref[...]).astype(jnp.float32)


def _run(x, W, b, g, *, tn):
    m, k = x.shape
    vocab = W.shape[1]
    nt = pl.cdiv(vocab, tn)
    b2 = b.reshape(1, vocab)
    kernel_fn = functools.partial(_gumbel_argmax_onehot, tn=tn, vocab=vocab)
    return pl.pallas_call(
        kernel_fn,
        grid=(2, nt),
        in_specs=[
            pl.BlockSpec((m, k), lambda p, j: (0, 0)),
            pl.BlockSpec((k, tn), lambda p, j: (0, jnp.where(p == 0, j, nt - 1))),
            pl.BlockSpec((1, tn), lambda p, j: (0, jnp.where(p == 0, j, nt - 1))),
            pl.BlockSpec((m, tn), lambda p, j: (0, jnp.where(p == 0, j, nt - 1))),
        ],
        out_specs=pl.BlockSpec((m, tn), lambda p, j: (0, jnp.where(p == 0, 0, j))),
        out_shape=jax.ShapeDtypeStruct((m, vocab), jnp.float32),
        scratch_shapes=[
            pltpu.VMEM((m, 1), jnp.float32),
            pltpu.VMEM((m, 1), jnp.int32),
        ],
        compiler_params=pltpu.CompilerParams(
            dimension_semantics=("arbitrary", "arbitrary"),
        ),
    )(x, W, b2, g)


def kernel(x, W, b):
    g = jax.random.gumbel(jax.random.key(_GUMBEL_SEED),
                          (x.shape[0], W.shape[1]), dtype=jnp.float32)
    return _run(x, W, b, g, tn=1024)


# speedup vs baseline: 1.0691x; 1.0691x over previous
import functools

import jax
import jax.numpy as jnp
import numpy as np
from jax.experimental import pallas as pl
from jax.experimental.pallas import tpu as pltpu

_KEY_HI = np.uint32(0)
_KEY_LO = np.uint32(1234)
_INT_MAX = 2**31 - 1


def _rotl(x, r):
    return (x << np.uint32(r)) | (x >> np.uint32(32 - r))


def _threefry_bits(lo):
    ks0 = _KEY_HI
    ks1 = _KEY_LO
    ks2 = np.uint32(0x1BD11BDA) ^ ks0 ^ ks1
    ks = (ks0, ks1, ks2)
    rotations = ((13, 15, 26, 6), (17, 29, 16, 24))
    x0 = jnp.zeros_like(lo) + ks0
    x1 = lo + ks1
    for i in range(5):
        for r in rotations[i % 2]:
            x0 = x0 + x1
            x1 = _rotl(x1, r)
            x1 = x1 ^ x0
        x0 = x0 + ks[(i + 1) % 3]
        x1 = x1 + ks[(i + 2) % 3] + np.uint32(i + 1)
    return x0 ^ x1


def _bits_to_gumbel(bits):
    fb = (bits >> np.uint32(9)) | np.uint32(0x3F800000)
    u = jax.lax.bitcast_convert_type(fb, jnp.float32) - np.float32(1.0)
    tiny = np.float32(np.finfo(np.float32).tiny)
    u = jnp.maximum(tiny, u * (np.float32(1.0) - tiny) + tiny)
    return -jnp.log(-jnp.log(u))


def _scan_kernel(x_ref, w_ref, b_ref, idx_ref, rmax_ref, *, tn, vocab):
    j = pl.program_id(0)
    m = x_ref.shape[0]
    logits = jnp.dot(x_ref[...], w_ref[...], preferred_element_type=jnp.float32)
    col = jax.lax.broadcasted_iota(jnp.int32, (m, tn), 1) + j * tn
    row = jax.lax.broadcasted_iota(jnp.uint32, (m, tn), 0)
    lo = row * np.uint32(vocab) + col.astype(jnp.uint32)
    g = _bits_to_gumbel(_threefry_bits(lo))
    vals = logits + b_ref[...] + g
    vals = jnp.where(col < vocab, vals, -jnp.inf)
    local_max = jnp.max(vals, axis=1, keepdims=True)
    cand = jnp.where(vals == local_max, col, _INT_MAX)
    local_arg = jnp.min(cand, axis=1, keepdims=True)

    @pl.when(j == 0)
    def _init():
        rmax_ref[...] = local_max
        idx_ref[...] = local_arg

    @pl.when(j > 0)
    def _update():
        better = local_max > rmax_ref[...]
        rmax_ref[...] = jnp.where(better, local_max, rmax_ref[...])
        idx_ref[...] = jnp.where(better, local_arg, idx_ref[...])


def _onehot_kernel(idx_ref, out_ref, *, tn):
    j = pl.program_id(0)
    m = out_ref.shape[0]
    col = jax.lax.broadcasted_iota(jnp.int32, (m, tn), 1) + j * tn
    out_ref[...] = (col == idx_ref[...]).astype(jnp.float32)


def _run(x, W, b, *, tn, tn2):
    m, k = x.shape
    vocab = W.shape[1]
    nt = pl.cdiv(vocab, tn)
    b2 = b.reshape(1, vocab)
    idx = pl.pallas_call(
        functools.partial(_scan_kernel, tn=tn, vocab=vocab),
        grid=(nt,),
        in_specs=[
            pl.BlockSpec((m, k), lambda j: (0, 0)),
            pl.BlockSpec((k, tn), lambda j: (0, j)),
            pl.BlockSpec((1, tn), lambda j: (0, j)),
        ],
        out_specs=pl.BlockSpec((m, 1), lambda j: (0, 0)),
        out_shape=jax.ShapeDtypeStruct((m, 1), jnp.int32),
        scratch_shapes=[pltpu.VMEM((m, 1), jnp.float32)],
        compiler_params=pltpu.CompilerParams(
            dimension_semantics=("arbitrary",),
        ),
    )(x, W, b2)
    nt2 = pl.cdiv(vocab, tn2)
    out = pl.pallas_call(
        functools.partial(_onehot_kernel, tn=tn2),
        grid=(nt2,),
        in_specs=[pl.BlockSpec((m, 1), lambda j: (0, 0))],
        out_specs=pl.BlockSpec((m, tn2), lambda j: (0, j)),
        out_shape=jax.ShapeDtypeStruct((m, vocab), jnp.float32),
        compiler_params=pltpu.CompilerParams(
            dimension_semantics=("arbitrary",),
        ),
    )(idx)
    return out


def kernel(x, W, b):
    return _run(x, W, b, tn=1024, tn2=4096)
